# baseline (speedup 1.0000x reference)
import functools

import numpy as np
import jax
import jax.numpy as jnp
from jax import lax
from jax.experimental import pallas as pl
from jax.experimental.pallas import tpu as pltpu
from jax.experimental.pallas import tpu_sc as plsc

_NUM_SPH = 7
_NUM_RAD = 6
_NCOL = _NUM_SPH * _NUM_RAD
_INV_CUTOFF = np.float32(1.0 / 5.0)

_NC = 2
_NS = 16
_NW = _NC * _NS
_CHUNK = 128


def _sph_jn_np(x, l):
    j0 = np.sin(x) / x
    if l == 0:
        return j0
    j1 = np.sin(x) / x ** 2 - np.cos(x) / x
    if l == 1:
        return j1
    jm1, j = j0, j1
    for n in range(1, l):
        jm1, j = j, (2 * n + 1) / x * j - jm1
    return j


def _jn_zeros_np(n, k):
    zerosj = np.zeros((n, k))
    zerosj[0] = np.arange(1, k + 1) * np.pi
    points = np.arange(1, k + n) * np.pi
    for i in range(1, n):
        m_roots = k + n - 1 - i
        racines = np.zeros(m_roots)
        for j in range(m_roots):
            a, b = points[j], points[j + 1]
            fa = _sph_jn_np(a, i)
            for _ in range(80):
                mid = 0.5 * (a + b)
                fm = _sph_jn_np(mid, i)
                if fa * fm <= 0:
                    b = mid
                else:
                    a = mid
                    fa = fm
            racines[j] = 0.5 * (a + b)
        points = racines
        zerosj[i] = racines[:k]
    return zerosj


_ZEROS64 = _jn_zeros_np(_NUM_SPH, _NUM_RAD)
_NORM32 = np.array(
    [[1.0 / np.sqrt(0.5 * _sph_jn_np(_ZEROS64[l, i], l + 1) ** 2)
      for i in range(_NUM_RAD)] for l in range(_NUM_SPH)],
    dtype=np.float32,
)
_ZROW = (_ZEROS64.astype(np.float32) * _INV_CUTOFF).astype(np.float32).reshape(1, _NCOL)
_NROW = _NORM32.reshape(1, _NCOL)
_LCOL = (np.arange(_NCOL, dtype=np.int32) // _NUM_RAD).reshape(1, _NCOL)
_LEG_COEF = [float(np.sqrt((2 * l + 1) / (4 * np.pi))) for l in range(_NUM_SPH)]


@functools.lru_cache(maxsize=None)
def _make_sc_gather(E, C):
    mesh = plsc.VectorSubcoreMesh(core_axis_name="c", subcore_axis_name="s")

    @functools.partial(
        pl.kernel,
        mesh=mesh,
        out_type=jax.ShapeDtypeStruct((_NW, C, _CHUNK), jnp.float32),
        scratch_types=[
            pltpu.VMEM((C, _CHUNK), jnp.int32),
            pltpu.VMEM((C, _CHUNK), jnp.float32),
            pltpu.SemaphoreType.DMA,
        ],
    )
    def gather_k(table_hbm, idx_hbm, out_hbm, idx_v, rows_v, sem):
        wid = lax.axis_index("s") * _NC + lax.axis_index("c")
        pltpu.sync_copy(idx_hbm.at[wid], idx_v)

        def group(g, carry):
            cps = []
            for b in range(8):
                j = g * 8 + b
                cps.append(
                    pltpu.async_copy(table_hbm.at[idx_v.at[j]], rows_v.at[j], sem))
            for cp in cps:
                cp.wait()
            return carry

        lax.fori_loop(0, C // 8, group, 0)
        pltpu.sync_copy(rows_v, out_hbm.at[wid])

    return gather_k


def _sc_gather(d, idx):
    T = idx.shape[0]
    E = d.shape[0]
    per_w = -(-T // (_NW * _CHUNK))
    C = -(-per_w // 8) * 8
    Tpad = _NW * C * _CHUNK
    if Tpad != T:
        idx = jnp.concatenate([idx, jnp.zeros((Tpad - T,), jnp.int32)])
    idx3 = idx.reshape(_NW, C, _CHUNK)
    out = _make_sc_gather(E, C)(d, idx3)
    out = out.reshape(Tpad)
    if Tpad != T:
        out = out[:T]
    return out


def _basis_body(dg_ref, ang_ref, zrow_ref, nrow_ref, lcol_ref, o_ref):
    zrow = zrow_ref[...]
    nrow = nrow_ref[...]
    lcol = lcol_ref[...]

    x = dg_ref[...] * _INV_CUTOFF
    X = zrow * dg_ref[...]
    s = jnp.sin(X)
    c = jnp.cos(X)
    j0 = s / X
    j1 = s / (X * X) - c / X
    jm1, j = j0, j1
    for m in range(1, _NUM_SPH - 1):
        jnew = (2 * m + 1) / X * j - jm1
        act = lcol >= (m + 1)
        jm1 = jnp.where(act, j, jm1)
        j = jnp.where(act, jnew, j)
    rb = jnp.where(lcol == 0, j0, j)
    rbf = nrow * rb

    env = 1.0 / x + (-36.0) * x ** 6 + 63.0 * x ** 7 + (-28.0) * x ** 8
    env = jnp.where(x < 10, env, jnp.zeros_like(env))
    rbf_env = env * rbf

    ct = jnp.cos(ang_ref[...])
    Ps = [jnp.ones_like(ct), ct]
    for l in range(2, _NUM_SPH):
        Ps.append(((2 * l - 1) * ct * Ps[l - 1] - (l - 1) * Ps[l - 2]) / l)
    cb = _LEG_COEF[_NUM_SPH - 1] * Ps[_NUM_SPH - 1]
    for l in range(_NUM_SPH - 2, -1, -1):
        cb = jnp.where(lcol == l, _LEG_COEF[l] * Ps[l], cb)

    o_ref[...] = rbf_env * cb


@functools.lru_cache(maxsize=None)
def _make_tc_basis(T, Bt):
    grid = (T // Bt,)
    return pl.pallas_call(
        _basis_body,
        grid=grid,
        in_specs=[
            pl.BlockSpec((Bt, 1), lambda i: (i, 0)),
            pl.BlockSpec((Bt, 1), lambda i: (i, 0)),
            pl.BlockSpec((1, _NCOL), lambda i: (0, 0)),
            pl.BlockSpec((1, _NCOL), lambda i: (0, 0)),
            pl.BlockSpec((1, _NCOL), lambda i: (0, 0)),
        ],
        out_specs=pl.BlockSpec((Bt, _NCOL), lambda i: (i, 0)),
        out_shape=jax.ShapeDtypeStruct((T, _NCOL), jnp.float32),
    )


def _pick_bt(T):
    for bt in (1600, 3200, 800, 1024, 2048, 512, 400, 200, 100, 8, 1):
        if T % bt == 0:
            return bt
    return 1


def kernel(d, Angles, id_expand_kj):
    T = id_expand_kj.shape[0]
    dg = _sc_gather(d, id_expand_kj)
    Bt = _pick_bt(T)
    out = _make_tc_basis(T, Bt)(
        dg.reshape(T, 1), Angles.reshape(T, 1),
        jnp.asarray(_ZROW), jnp.asarray(_NROW), jnp.asarray(_LCOL))
    return out


# speedup vs baseline: 1.5510x; 1.5510x over previous
import functools

import numpy as np
import jax
import jax.numpy as jnp
from jax import lax
from jax.experimental import pallas as pl
from jax.experimental.pallas import tpu as pltpu
from jax.experimental.pallas import tpu_sc as plsc

_NUM_SPH = 7
_NUM_RAD = 6
_NCOL = _NUM_SPH * _NUM_RAD
_INV_CUTOFF = np.float32(1.0 / 5.0)

_NC = 2
_NS = 16
_NW = _NC * _NS
_CHUNK = 128


def _sph_jn_np(x, l):
    j0 = np.sin(x) / x
    if l == 0:
        return j0
    j1 = np.sin(x) / x ** 2 - np.cos(x) / x
    if l == 1:
        return j1
    jm1, j = j0, j1
    for n in range(1, l):
        jm1, j = j, (2 * n + 1) / x * j - jm1
    return j


def _jn_zeros_np(n, k):
    zerosj = np.zeros((n, k))
    zerosj[0] = np.arange(1, k + 1) * np.pi
    points = np.arange(1, k + n) * np.pi
    for i in range(1, n):
        m_roots = k + n - 1 - i
        racines = np.zeros(m_roots)
        for j in range(m_roots):
            a, b = points[j], points[j + 1]
            fa = _sph_jn_np(a, i)
            for _ in range(80):
                mid = 0.5 * (a + b)
                fm = _sph_jn_np(mid, i)
                if fa * fm <= 0:
                    b = mid
                else:
                    a = mid
                    fa = fm
            racines[j] = 0.5 * (a + b)
        points = racines
        zerosj[i] = racines[:k]
    return zerosj


_ZEROS64 = _jn_zeros_np(_NUM_SPH, _NUM_RAD)
_NORM32 = np.array(
    [[1.0 / np.sqrt(0.5 * _sph_jn_np(_ZEROS64[l, i], l + 1) ** 2)
      for i in range(_NUM_RAD)] for l in range(_NUM_SPH)],
    dtype=np.float32,
)
_ZROW = (_ZEROS64.astype(np.float32) * _INV_CUTOFF).astype(np.float32).reshape(1, _NCOL)
_NROW = _NORM32.reshape(1, _NCOL)
_LCOL = (np.arange(_NCOL, dtype=np.int32) // _NUM_RAD).reshape(1, _NCOL)
_LEG_COEF = [float(np.sqrt((2 * l + 1) / (4 * np.pi))) for l in range(_NUM_SPH)]


@functools.lru_cache(maxsize=None)
def _make_sc_gather(E, C):
    mesh = plsc.VectorSubcoreMesh(core_axis_name="c", subcore_axis_name="s")

    @functools.partial(
        pl.kernel,
        mesh=mesh,
        out_type=jax.ShapeDtypeStruct((_NW, C, _CHUNK), jnp.float32),
        scratch_types=[
            pltpu.VMEM((C, _CHUNK), jnp.int32),
            pltpu.VMEM((C, _CHUNK), jnp.float32),
            pltpu.SemaphoreType.DMA,
        ],
    )
    def gather_k(table_hbm, idx_hbm, out_hbm, idx_v, rows_v, sem):
        wid = lax.axis_index("s") * _NC + lax.axis_index("c")
        pltpu.sync_copy(idx_hbm.at[wid], idx_v)

        def group(g, carry):
            cps = []
            for b in range(8):
                j = g * 8 + b
                cps.append(
                    pltpu.async_copy(table_hbm.at[idx_v.at[j]], rows_v.at[j], sem))
            for cp in cps:
                cp.wait()
            return carry

        lax.fori_loop(0, C // 8, group, 0)
        pltpu.sync_copy(rows_v, out_hbm.at[wid])

    return gather_k


def _sc_gather(d, idx):
    T = idx.shape[0]
    E = d.shape[0]
    per_w = -(-T // (_NW * _CHUNK))
    C = -(-per_w // 8) * 8
    Tpad = _NW * C * _CHUNK
    if Tpad != T:
        idx = jnp.concatenate([idx, jnp.zeros((Tpad - T,), jnp.int32)])
    idx3 = idx.reshape(_NW, C, _CHUNK)
    out = _make_sc_gather(E, C)(d, idx3)
    out = out.reshape(Tpad)
    if Tpad != T:
        out = out[:T]
    return out


def _make_basis_body(P):
    W = _NCOL * P

    def body(dg_ref, ang_ref, zrow_ref, nrow_ref, lcol_ref, gcol_ref, o_ref):
        zrow = zrow_ref[...]
        nrow = nrow_ref[...]
        lcol = lcol_ref[...]
        gcol = gcol_ref[...]
        dgc = dg_ref[...]
        angc = ang_ref[...]

        def lane_route(mat):
            v = mat[:, P - 1:P]
            for k in range(P - 2, -1, -1):
                v = jnp.where(gcol == k, mat[:, k:k + 1], v)
            return v

        dsel = lane_route(dgc)
        X = zrow * dsel
        s = jnp.sin(X)
        c = jnp.cos(X)
        j0 = s / X
        j1 = s / (X * X) - c / X
        jm1, j = j0, j1
        for m in range(1, _NUM_SPH - 1):
            jnew = (2 * m + 1) / X * j - jm1
            act = lcol >= (m + 1)
            jm1 = jnp.where(act, j, jm1)
            j = jnp.where(act, jnew, j)
        rb = jnp.where(lcol == 0, j0, j)
        rbf = nrow * rb

        x = dgc * _INV_CUTOFF
        env = 1.0 / x + (-36.0) * x ** 6 + 63.0 * x ** 7 + (-28.0) * x ** 8
        env = jnp.where(x < 10, env, jnp.zeros_like(env))
        rbf_env = lane_route(env) * rbf

        ct = jnp.cos(angc)
        Ps = [jnp.ones_like(ct), ct]
        for l in range(2, _NUM_SPH):
            Ps.append(((2 * l - 1) * ct * Ps[l - 1] - (l - 1) * Ps[l - 2]) / l)
        scaled = [lane_route(_LEG_COEF[l] * Ps[l]) for l in range(_NUM_SPH)]
        cb = scaled[_NUM_SPH - 1]
        for l in range(_NUM_SPH - 2, -1, -1):
            cb = jnp.where(lcol == l, scaled[l], cb)

        o_ref[...] = rbf_env * cb

    return body


@functools.lru_cache(maxsize=None)
def _make_tc_basis(T, Bt, P):
    R = T // P
    W = _NCOL * P
    grid = (R // Bt,)
    return pl.pallas_call(
        _make_basis_body(P),
        grid=grid,
        in_specs=[
            pl.BlockSpec((Bt, P), lambda i: (i, 0)),
            pl.BlockSpec((Bt, P), lambda i: (i, 0)),
            pl.BlockSpec((1, W), lambda i: (0, 0)),
            pl.BlockSpec((1, W), lambda i: (0, 0)),
            pl.BlockSpec((1, W), lambda i: (0, 0)),
            pl.BlockSpec((1, W), lambda i: (0, 0)),
        ],
        out_specs=pl.BlockSpec((Bt, W), lambda i: (i, 0)),
        out_shape=jax.ShapeDtypeStruct((R, W), jnp.float32),
    )


def _pick_layout(T):
    for P in (3, 2, 1):
        if T % P:
            continue
        R = T // P
        for bt in (1600, 3200, 800, 1024, 2048, 512, 400, 250, 200, 125, 100, 8, 1):
            if R % bt == 0 and bt % 8 == 0:
                return P, bt
    return 1, 1


def kernel(d, Angles, id_expand_kj):
    T = id_expand_kj.shape[0]
    dg = _sc_gather(d, id_expand_kj)
    P, Bt = _pick_layout(T)
    R = T // P
    W = _NCOL * P
    zrow = jnp.asarray(np.tile(_ZROW, (1, P)))
    nrow = jnp.asarray(np.tile(_NROW, (1, P)))
    lcol = jnp.asarray(np.tile(_LCOL, (1, P)))
    gcol = jnp.asarray((np.arange(W, dtype=np.int32) // _NCOL).reshape(1, W))
    out = _make_tc_basis(T, Bt, P)(
        dg.reshape(R, P), Angles.reshape(R, P), zrow, nrow, lcol, gcol)
    return out.reshape(T, _NCOL)


# speedup vs baseline: 1.6145x; 1.0409x over previous
import functools

import numpy as np
import jax
import jax.numpy as jnp
from jax import lax
from jax.experimental import pallas as pl
from jax.experimental.pallas import tpu as pltpu
from jax.experimental.pallas import tpu_sc as plsc

_NUM_SPH = 7
_NUM_RAD = 6
_NCOL = _NUM_SPH * _NUM_RAD
_INV_CUTOFF = np.float32(1.0 / 5.0)

_NC = 2
_NS = 16
_NW = _NC * _NS
_CHUNK = 128


def _sph_jn_np(x, l):
    j0 = np.sin(x) / x
    if l == 0:
        return j0
    j1 = np.sin(x) / x ** 2 - np.cos(x) / x
    if l == 1:
        return j1
    jm1, j = j0, j1
    for n in range(1, l):
        jm1, j = j, (2 * n + 1) / x * j - jm1
    return j


def _jn_zeros_np(n, k):
    zerosj = np.zeros((n, k))
    zerosj[0] = np.arange(1, k + 1) * np.pi
    points = np.arange(1, k + n) * np.pi
    for i in range(1, n):
        m_roots = k + n - 1 - i
        racines = np.zeros(m_roots)
        for j in range(m_roots):
            a, b = points[j], points[j + 1]
            fa = _sph_jn_np(a, i)
            for _ in range(80):
                mid = 0.5 * (a + b)
                fm = _sph_jn_np(mid, i)
                if fa * fm <= 0:
                    b = mid
                else:
                    a = mid
                    fa = fm
            racines[j] = 0.5 * (a + b)
        points = racines
        zerosj[i] = racines[:k]
    return zerosj


_ZEROS64 = _jn_zeros_np(_NUM_SPH, _NUM_RAD)
_NORM32 = np.array(
    [[1.0 / np.sqrt(0.5 * _sph_jn_np(_ZEROS64[l, i], l + 1) ** 2)
      for i in range(_NUM_RAD)] for l in range(_NUM_SPH)],
    dtype=np.float32,
)
_ZROW = (_ZEROS64.astype(np.float32) * _INV_CUTOFF).astype(np.float32).reshape(1, _NCOL)
_NROW = _NORM32.reshape(1, _NCOL)
_LCOL = (np.arange(_NCOL, dtype=np.int32) // _NUM_RAD).reshape(1, _NCOL)
_LEG_COEF = [float(np.sqrt((2 * l + 1) / (4 * np.pi))) for l in range(_NUM_SPH)]


@functools.lru_cache(maxsize=None)
def _make_sc_gather(E, C):
    mesh = plsc.VectorSubcoreMesh(core_axis_name="c", subcore_axis_name="s")

    @functools.partial(
        pl.kernel,
        mesh=mesh,
        out_type=jax.ShapeDtypeStruct((_NW, C, _CHUNK), jnp.float32),
        scratch_types=[
            pltpu.VMEM((C, _CHUNK), jnp.int32),
            pltpu.VMEM((C, _CHUNK), jnp.float32),
            pltpu.SemaphoreType.DMA,
        ],
    )
    def gather_k(table_hbm, idx_hbm, out_hbm, idx_v, rows_v, sem):
        wid = lax.axis_index("s") * _NC + lax.axis_index("c")
        pltpu.sync_copy(idx_hbm.at[wid], idx_v)

        def group(g, carry):
            cps = []
            for b in range(8):
                j = g * 8 + b
                cps.append(
                    pltpu.async_copy(table_hbm.at[idx_v.at[j]], rows_v.at[j], sem))
            for cp in cps:
                cp.wait()
            return carry

        lax.fori_loop(0, C // 8, group, 0)
        pltpu.sync_copy(rows_v, out_hbm.at[wid])

    return gather_k


def _sc_gather(d, idx):
    T = idx.shape[0]
    E = d.shape[0]
    per_w = -(-T // (_NW * _CHUNK))
    C = -(-per_w // 8) * 8
    Tpad = _NW * C * _CHUNK
    if Tpad != T:
        idx = jnp.concatenate([idx, jnp.zeros((Tpad - T,), jnp.int32)])
    idx3 = idx.reshape(_NW, C, _CHUNK)
    out = _make_sc_gather(E, C)(d, idx3)
    return out.reshape(Tpad)


def _make_basis_body(P):
    W = _NCOL * P

    def body(dg_ref, ang_ref, zrow_ref, nrow_ref, lcol_ref, gcol_ref, o_ref):
        zrow = zrow_ref[...]
        nrow = nrow_ref[...]
        lcol = lcol_ref[...]
        gcol = gcol_ref[...]
        dgc = dg_ref[...]
        angc = ang_ref[...]

        def lane_route(mat):
            v = mat[:, P - 1:P]
            for k in range(P - 2, -1, -1):
                v = jnp.where(gcol == k, mat[:, k:k + 1], v)
            return v

        dsel = lane_route(dgc)
        X = zrow * dsel
        s = jnp.sin(X)
        c = jnp.cos(X)
        j0 = s / X
        j1 = s / (X * X) - c / X
        jm1, j = j0, j1
        for m in range(1, _NUM_SPH - 1):
            jnew = (2 * m + 1) / X * j - jm1
            act = lcol >= (m + 1)
            jm1 = jnp.where(act, j, jm1)
            j = jnp.where(act, jnew, j)
        rb = jnp.where(lcol == 0, j0, j)
        rbf = nrow * rb

        x = dgc * _INV_CUTOFF
        env = 1.0 / x + (-36.0) * x ** 6 + 63.0 * x ** 7 + (-28.0) * x ** 8
        env = jnp.where(x < 10, env, jnp.zeros_like(env))
        rbf_env = lane_route(env) * rbf

        ct = jnp.cos(angc)
        Ps = [jnp.ones_like(ct), ct]
        for l in range(2, _NUM_SPH):
            Ps.append(((2 * l - 1) * ct * Ps[l - 1] - (l - 1) * Ps[l - 2]) / l)
        scaled = [lane_route(_LEG_COEF[l] * Ps[l]) for l in range(_NUM_SPH)]
        cb = scaled[_NUM_SPH - 1]
        for l in range(_NUM_SPH - 2, -1, -1):
            cb = jnp.where(lcol == l, scaled[l], cb)

        o_ref[...] = rbf_env * cb

    return body


@functools.lru_cache(maxsize=None)
def _make_tc_basis(T, Bt, P):
    R = T // P
    W = _NCOL * P
    grid = (R // Bt,)
    return pl.pallas_call(
        _make_basis_body(P),
        grid=grid,
        in_specs=[
            pl.BlockSpec((Bt, P), lambda i: (i, 0)),
            pl.BlockSpec((Bt, P), lambda i: (i, 0)),
            pl.BlockSpec((1, W), lambda i: (0, 0)),
            pl.BlockSpec((1, W), lambda i: (0, 0)),
            pl.BlockSpec((1, W), lambda i: (0, 0)),
            pl.BlockSpec((1, W), lambda i: (0, 0)),
        ],
        out_specs=pl.BlockSpec((Bt, W), lambda i: (i, 0)),
        out_shape=jax.ShapeDtypeStruct((R, W), jnp.float32),
    )


def _pick_layout(T):
    for P in (3, 2, 1):
        if T % P:
            continue
        R = T // P
        for bt in (1600, 3200, 800, 1024, 2048, 512, 400, 250, 200, 125, 100, 8, 1):
            if R % bt == 0 and bt % 8 == 0:
                return P, bt
    return 1, 1


def kernel(d, Angles, id_expand_kj):
    T = id_expand_kj.shape[0]
    dg = _sc_gather(d, id_expand_kj)
    P, Bt = _pick_layout(T)
    R = T // P
    W = _NCOL * P
    zrow = jnp.asarray(np.tile(_ZROW, (1, P)))
    nrow = jnp.asarray(np.tile(_NROW, (1, P)))
    lcol = jnp.asarray(np.tile(_LCOL, (1, P)))
    gcol = jnp.asarray((np.arange(W, dtype=np.int32) // _NCOL).reshape(1, W))
    dgp = dg.reshape(dg.shape[0] // P, P)
    out = _make_tc_basis(T, Bt, P)(
        dgp, Angles.reshape(R, P), zrow, nrow, lcol, gcol)
    return out.reshape(T, _NCOL)
